# Initial kernel scaffold; baseline (speedup 1.0000x reference)
import jax
import jax.numpy as jnp
from jax.experimental import pallas as pl


def kernel(school_idx, grade_idx, goal_idx, subject_idx, method_idx, school_table, grade_table, goal_table, subject_table, method_table, W1, b1, W2, b2, W3, b3):
    raise NotImplementedError("write your pallas kernel here")


# speedup vs baseline: 3.0130x; 3.0130x over previous
import functools

import jax
import jax.numpy as jnp
from jax import lax
from jax.experimental import pallas as pl
from jax.experimental.pallas import tpu as pltpu
from jax.experimental.pallas import tpu_sc as plsc

_GATHER_CHUNK = 128


def _sc_gather(table, idx):
    B = idx.shape[0]
    D = table.shape[1]
    info = plsc.get_sparse_core_info()
    nw = info.num_cores * info.num_subcores
    b_per_w = B // nw
    mesh = plsc.VectorSubcoreMesh(core_axis_name="c", subcore_axis_name="s")

    @functools.partial(
        pl.kernel,
        mesh=mesh,
        compiler_params=pltpu.CompilerParams(use_tc_tiling_on_sc=False),
        out_type=jax.ShapeDtypeStruct((B, D), jnp.float32),
        scratch_types=[
            pltpu.VMEM((b_per_w,), jnp.int32),
            pltpu.VMEM((b_per_w, D), jnp.float32),
            pltpu.SemaphoreType.DMA,
        ],
    )
    def gather_kernel(table_hbm, idx_hbm, out_hbm, idx_v, rows_v, sem):
        wid = lax.axis_index("s") * info.num_cores + lax.axis_index("c")
        base = wid * b_per_w
        pltpu.sync_copy(idx_hbm.at[pl.ds(base, b_per_w)], idx_v)
        copies = []
        for j in range(0, b_per_w, _GATHER_CHUNK):
            n = min(_GATHER_CHUNK, b_per_w - j)
            copies.append(
                pltpu.async_copy(
                    table_hbm.at[idx_v.at[pl.ds(j, n)]],
                    rows_v.at[pl.ds(j, n)],
                    sem,
                )
            )
        for c in copies:
            c.wait()
        pltpu.sync_copy(rows_v, out_hbm.at[pl.ds(base, b_per_w)])

    return gather_kernel(table, idx)


def _mlp_body(sch_ref, g_ref, go_ref, su_ref, me_ref,
              gt_ref, got_ref, sut_ref, met_ref,
              w1_ref, b1_ref, w2_ref, b2_ref, w3_ref, b3_ref, out_ref):
    bk = sch_ref.shape[0]

    def emb(idx_ref, tab_ref):
        v = tab_ref.shape[0]
        oh = (idx_ref[...] == lax.broadcasted_iota(jnp.int32, (bk, v), 1))
        return jnp.dot(oh.astype(jnp.float32), tab_ref[...],
                       preferred_element_type=jnp.float32)

    x = jnp.concatenate(
        [sch_ref[...], emb(g_ref, gt_ref), emb(go_ref, got_ref),
         emb(su_ref, sut_ref), emb(me_ref, met_ref)], axis=1)
    h = jnp.maximum(
        jnp.dot(x, w1_ref[...], preferred_element_type=jnp.float32)
        + b1_ref[...], 0.0)
    h = jnp.maximum(
        jnp.dot(h, w2_ref[...], preferred_element_type=jnp.float32)
        + b2_ref[...], 0.0)
    out_ref[...] = (
        jnp.dot(h, w3_ref[...], preferred_element_type=jnp.float32)
        + b3_ref[...])


def _pad_rows(t):
    v = t.shape[0]
    vp = -(-v // 8) * 8
    return jnp.pad(t, ((0, vp - v), (0, 0)))


def _tc_mlp(sch_emb, g, go, su, me, gt, got, sut, met, W1, b1, W2, b2, W3, b3):
    B, D = sch_emb.shape
    BK = 2048
    grid = B // BK
    H1, H2, DO = W1.shape[1], W2.shape[1], W3.shape[1]

    def blk(i, *_):
        return (i, 0)

    def rep(*_):
        return (0, 0)

    return pl.pallas_call(
        _mlp_body,
        grid=(grid,),
        in_specs=[
            pl.BlockSpec((BK, D), blk),
            pl.BlockSpec((BK, 1), blk),
            pl.BlockSpec((BK, 1), blk),
            pl.BlockSpec((BK, 1), blk),
            pl.BlockSpec((BK, 1), blk),
            pl.BlockSpec(gt.shape, rep),
            pl.BlockSpec(got.shape, rep),
            pl.BlockSpec(sut.shape, rep),
            pl.BlockSpec(met.shape, rep),
            pl.BlockSpec(W1.shape, rep),
            pl.BlockSpec((1, H1), rep),
            pl.BlockSpec(W2.shape, rep),
            pl.BlockSpec((1, H2), rep),
            pl.BlockSpec(W3.shape, rep),
            pl.BlockSpec((1, DO), rep),
        ],
        out_specs=pl.BlockSpec((BK, DO), blk),
        out_shape=jax.ShapeDtypeStruct((B, DO), jnp.float32),
    )(sch_emb, g.reshape(B, 1), go.reshape(B, 1), su.reshape(B, 1),
      me.reshape(B, 1), gt, got, sut, met,
      W1, b1.reshape(1, H1), W2, b2.reshape(1, H2), W3, b3.reshape(1, DO))


def kernel(school_idx, grade_idx, goal_idx, subject_idx, method_idx,
           school_table, grade_table, goal_table, subject_table, method_table,
           W1, b1, W2, b2, W3, b3):
    sch_emb = _sc_gather(school_table, school_idx.astype(jnp.int32))
    return _tc_mlp(
        sch_emb,
        grade_idx.astype(jnp.int32), goal_idx.astype(jnp.int32),
        subject_idx.astype(jnp.int32), method_idx.astype(jnp.int32),
        _pad_rows(grade_table), _pad_rows(goal_table),
        _pad_rows(subject_table), _pad_rows(method_table),
        W1, b1, W2, b2, W3, b3)


# speedup vs baseline: 3.8122x; 1.2652x over previous
import functools

import jax
import jax.numpy as jnp
from jax import lax
from jax.experimental import pallas as pl
from jax.experimental.pallas import tpu as pltpu
from jax.experimental.pallas import tpu_sc as plsc

_GATHER_CHUNK = 128


def _sc_gather(table, idx):
    B = idx.shape[0]
    D = table.shape[1]
    info = plsc.get_sparse_core_info()
    nw = info.num_cores * info.num_subcores
    b_per_w = B // nw
    mesh = plsc.VectorSubcoreMesh(core_axis_name="c", subcore_axis_name="s")

    @functools.partial(
        pl.kernel,
        mesh=mesh,
        compiler_params=pltpu.CompilerParams(use_tc_tiling_on_sc=False),
        out_type=jax.ShapeDtypeStruct((B, D), jnp.float32),
        scratch_types=[
            pltpu.VMEM((b_per_w,), jnp.int32),
            pltpu.VMEM((b_per_w, D), jnp.float32),
            pltpu.SemaphoreType.DMA,
        ],
    )
    def gather_kernel(table_hbm, idx_hbm, out_hbm, idx_v, rows_v, sem):
        wid = lax.axis_index("s") * info.num_cores + lax.axis_index("c")
        base = wid * b_per_w
        pltpu.sync_copy(idx_hbm.at[pl.ds(base, b_per_w)], idx_v)
        copies = []
        for j in range(0, b_per_w, _GATHER_CHUNK):
            n = min(_GATHER_CHUNK, b_per_w - j)
            copies.append(
                pltpu.async_copy(
                    table_hbm.at[idx_v.at[pl.ds(j, n)]],
                    rows_v.at[pl.ds(j, n)],
                    sem,
                )
            )
        for c in copies:
            c.wait()
        pltpu.sync_copy(rows_v, out_hbm.at[pl.ds(base, b_per_w)])

    return gather_kernel(table, idx)


def _mlp_body(sch_ref, g_ref, go_ref, su_ref, me_ref,
              gt_ref, got_ref, sut_ref, met_ref,
              w1_ref, b1_ref, w2_ref, b2_ref, w3_ref, b3_ref, out_ref):
    bk = sch_ref.shape[0]

    def emb(idx_ref, tab_ref):
        v = tab_ref.shape[0]
        oh_t = (idx_ref[...][None, :] == lax.broadcasted_iota(jnp.int32, (v, bk), 0))
        return lax.dot_general(oh_t.astype(jnp.float32), tab_ref[...],
                               (((0,), (0,)), ((), ())),
                               preferred_element_type=jnp.float32)

    x = jnp.concatenate(
        [sch_ref[...], emb(g_ref, gt_ref), emb(go_ref, got_ref),
         emb(su_ref, sut_ref), emb(me_ref, met_ref)], axis=1)
    h = jnp.maximum(
        jnp.dot(x, w1_ref[...], preferred_element_type=jnp.float32)
        + b1_ref[...], 0.0)
    h = jnp.maximum(
        jnp.dot(h, w2_ref[...], preferred_element_type=jnp.float32)
        + b2_ref[...], 0.0)
    out_ref[...] = (
        jnp.dot(h, w3_ref[...], preferred_element_type=jnp.float32)
        + b3_ref[...])


def _pad_rows(t):
    v = t.shape[0]
    vp = -(-v // 8) * 8
    return jnp.pad(t, ((0, vp - v), (0, 0)))


def _tc_mlp(sch_emb, g, go, su, me, gt, got, sut, met, W1, b1, W2, b2, W3, b3):
    B, D = sch_emb.shape
    BK = 2048
    grid = B // BK
    H1, H2, DO = W1.shape[1], W2.shape[1], W3.shape[1]

    def blk(i, *_):
        return (i, 0)

    def blk1(i, *_):
        return (i,)

    def rep(*_):
        return (0, 0)

    return pl.pallas_call(
        _mlp_body,
        grid=(grid,),
        in_specs=[
            pl.BlockSpec((BK, D), blk),
            pl.BlockSpec((BK,), blk1),
            pl.BlockSpec((BK,), blk1),
            pl.BlockSpec((BK,), blk1),
            pl.BlockSpec((BK,), blk1),
            pl.BlockSpec(gt.shape, rep),
            pl.BlockSpec(got.shape, rep),
            pl.BlockSpec(sut.shape, rep),
            pl.BlockSpec(met.shape, rep),
            pl.BlockSpec(W1.shape, rep),
            pl.BlockSpec((1, H1), rep),
            pl.BlockSpec(W2.shape, rep),
            pl.BlockSpec((1, H2), rep),
            pl.BlockSpec(W3.shape, rep),
            pl.BlockSpec((1, DO), rep),
        ],
        out_specs=pl.BlockSpec((BK, DO), blk),
        out_shape=jax.ShapeDtypeStruct((B, DO), jnp.float32),
    )(sch_emb, g, go, su, me, gt, got, sut, met,
      W1, b1.reshape(1, H1), W2, b2.reshape(1, H2), W3, b3.reshape(1, DO))


def kernel(school_idx, grade_idx, goal_idx, subject_idx, method_idx,
           school_table, grade_table, goal_table, subject_table, method_table,
           W1, b1, W2, b2, W3, b3):
    sch_emb = _sc_gather(school_table, school_idx.astype(jnp.int32))
    return _tc_mlp(
        sch_emb,
        grade_idx.astype(jnp.int32), goal_idx.astype(jnp.int32),
        subject_idx.astype(jnp.int32), method_idx.astype(jnp.int32),
        _pad_rows(grade_table), _pad_rows(goal_table),
        _pad_rows(subject_table), _pad_rows(method_table),
        W1, b1, W2, b2, W3, b3)


# speedup vs baseline: 4.7539x; 1.2470x over previous
import functools

import jax
import jax.numpy as jnp
from jax import lax
from jax.experimental import pallas as pl
from jax.experimental.pallas import tpu as pltpu
from jax.experimental.pallas import tpu_sc as plsc

_GATHER_CHUNK = 128


def _sc_gather(table, idx):
    B = idx.shape[0]
    D = table.shape[1]
    info = plsc.get_sparse_core_info()
    nw = info.num_cores * info.num_subcores
    b_per_w = B // nw
    mesh = plsc.VectorSubcoreMesh(core_axis_name="c", subcore_axis_name="s")

    @functools.partial(
        pl.kernel,
        mesh=mesh,
        out_type=jax.ShapeDtypeStruct((B, D), jnp.float32),
        scratch_types=[
            pltpu.VMEM((b_per_w + 16,), jnp.int32),
            pltpu.VMEM((b_per_w, D), jnp.float32),
            pltpu.SemaphoreType.DMA,
        ],
    )
    def gather_kernel(table_hbm, idx_hbm, out_hbm, idx_v, rows_v, sem):
        wid = lax.axis_index("s") * info.num_cores + lax.axis_index("c")
        base = wid * b_per_w
        pltpu.sync_copy(idx_hbm.at[pl.ds(base, b_per_w)],
                        idx_v.at[pl.ds(0, b_per_w)])

        @pl.loop(0, b_per_w)
        def _(i):
            k = idx_v[pl.ds(i, 16)][0]
            pltpu.async_copy(table_hbm.at[pl.ds(k, 1)],
                             rows_v.at[pl.ds(i, 1)], sem)

        pltpu.make_async_copy(table_hbm.at[pl.ds(0, b_per_w)], rows_v,
                              sem).wait()
        pltpu.sync_copy(rows_v, out_hbm.at[pl.ds(base, b_per_w)])

    return gather_kernel(table, idx)


def _mlp_body(sch_ref, g_ref, go_ref, su_ref, me_ref,
              gt_ref, got_ref, sut_ref, met_ref,
              w1_ref, b1_ref, w2_ref, b2_ref, w3_ref, b3_ref, out_ref):
    bk = sch_ref.shape[0]

    def emb(idx_ref, tab_ref):
        v = tab_ref.shape[0]
        oh_t = (idx_ref[...][None, :] == lax.broadcasted_iota(jnp.int32, (v, bk), 0))
        return lax.dot_general(oh_t.astype(jnp.float32), tab_ref[...],
                               (((0,), (0,)), ((), ())),
                               preferred_element_type=jnp.float32)

    x = jnp.concatenate(
        [sch_ref[...], emb(g_ref, gt_ref), emb(go_ref, got_ref),
         emb(su_ref, sut_ref), emb(me_ref, met_ref)], axis=1)
    h = jnp.maximum(
        jnp.dot(x, w1_ref[...], preferred_element_type=jnp.float32)
        + b1_ref[...], 0.0)
    h = jnp.maximum(
        jnp.dot(h, w2_ref[...], preferred_element_type=jnp.float32)
        + b2_ref[...], 0.0)
    out_ref[...] = (
        jnp.dot(h, w3_ref[...], preferred_element_type=jnp.float32)
        + b3_ref[...])


def _pad_rows(t):
    v = t.shape[0]
    vp = -(-v // 8) * 8
    return jnp.pad(t, ((0, vp - v), (0, 0)))


def _tc_mlp(sch_emb, g, go, su, me, gt, got, sut, met, W1, b1, W2, b2, W3, b3):
    B, D = sch_emb.shape
    BK = 2048
    grid = B // BK
    H1, H2, DO = W1.shape[1], W2.shape[1], W3.shape[1]

    def blk(i, *_):
        return (i, 0)

    def blk1(i, *_):
        return (i,)

    def rep(*_):
        return (0, 0)

    return pl.pallas_call(
        _mlp_body,
        grid=(grid,),
        in_specs=[
            pl.BlockSpec((BK, D), blk),
            pl.BlockSpec((BK,), blk1),
            pl.BlockSpec((BK,), blk1),
            pl.BlockSpec((BK,), blk1),
            pl.BlockSpec((BK,), blk1),
            pl.BlockSpec(gt.shape, rep),
            pl.BlockSpec(got.shape, rep),
            pl.BlockSpec(sut.shape, rep),
            pl.BlockSpec(met.shape, rep),
            pl.BlockSpec(W1.shape, rep),
            pl.BlockSpec((1, H1), rep),
            pl.BlockSpec(W2.shape, rep),
            pl.BlockSpec((1, H2), rep),
            pl.BlockSpec(W3.shape, rep),
            pl.BlockSpec((1, DO), rep),
        ],
        out_specs=pl.BlockSpec((BK, DO), blk),
        out_shape=jax.ShapeDtypeStruct((B, DO), jnp.float32),
    )(sch_emb, g, go, su, me, gt, got, sut, met,
      W1, b1.reshape(1, H1), W2, b2.reshape(1, H2), W3, b3.reshape(1, DO))


def kernel(school_idx, grade_idx, goal_idx, subject_idx, method_idx,
           school_table, grade_table, goal_table, subject_table, method_table,
           W1, b1, W2, b2, W3, b3):
    sch_emb = _sc_gather(school_table, school_idx.astype(jnp.int32))
    return _tc_mlp(
        sch_emb,
        grade_idx.astype(jnp.int32), goal_idx.astype(jnp.int32),
        subject_idx.astype(jnp.int32), method_idx.astype(jnp.int32),
        _pad_rows(grade_table), _pad_rows(goal_table),
        _pad_rows(subject_table), _pad_rows(method_table),
        W1, b1, W2, b2, W3, b3)
